# baseline (speedup 1.0000x reference)
import functools

import jax
import jax.numpy as jnp
from jax import lax
from jax.experimental import pallas as pl
from jax.experimental.pallas import tpu as pltpu
from jax.experimental.pallas import tpu_sc as plsc

D_MODEL = 128
BATCH = 16384
SCALE = 0.2

_INFO = plsc.get_sparse_core_info()
_NC = _INFO.num_cores
_NS = _INFO.num_subcores
_LANES = _INFO.num_lanes
_NW = _NC * _NS
_B_PER_W = BATCH // _NW
_CHUNK = 32
_N_CHUNK = _B_PER_W // _CHUNK
_VPR = D_MODEL // _LANES


def _make_sc_gather():
    mesh = plsc.VectorSubcoreMesh(core_axis_name="c", subcore_axis_name="s")

    @functools.partial(
        pl.kernel,
        mesh=mesh,
        out_type=jax.ShapeDtypeStruct((BATCH, D_MODEL), jnp.float32),
        scratch_types=[
            pltpu.VMEM((_B_PER_W,), jnp.int32),
            pltpu.VMEM((_N_CHUNK, _CHUNK, D_MODEL), jnp.float32),
        ]
        + [pltpu.SemaphoreType.DMA] * (_N_CHUNK + 1),
    )
    def sc_gather(table_hbm, idx_hbm, out_hbm, idx_v, rows_v, *sems):
        gsems, ssem = sems[:_N_CHUNK], sems[_N_CHUNK]
        wid = lax.axis_index("s") * _NC + lax.axis_index("c")
        base = wid * _B_PER_W
        pltpu.sync_copy(idx_hbm.at[pl.ds(wid * _B_PER_W, _B_PER_W)], idx_v)
        gathers = [
            pltpu.async_copy(
                table_hbm.at[idx_v.at[pl.ds(j * _CHUNK, _CHUNK)]],
                rows_v.at[j],
                gsems[j],
            )
            for j in range(_N_CHUNK)
        ]
        stores = []
        for j in range(_N_CHUNK):
            gathers[j].wait()

            def scale_rows(r, _, j=j):
                for rr in range(2):
                    for c in range(_VPR):
                        sl = pl.ds(c * _LANES, _LANES)
                        rows_v[j, r * 2 + rr, sl] = rows_v[j, r * 2 + rr, sl] * SCALE
                return _

            lax.fori_loop(0, _CHUNK // 2, scale_rows, None)
            stores.append(
                pltpu.async_copy(
                    rows_v.at[j], out_hbm.at[pl.ds(base + j * _CHUNK, _CHUNK)], ssem
                )
            )
        for s in stores:
            s.wait()

    return sc_gather


_SC_GATHER = _make_sc_gather()


def kernel(pe, t):
    return _SC_GATHER(pe, t)


# speedup vs baseline: 1.0310x; 1.0310x over previous
import functools

import jax
import jax.numpy as jnp
from jax import lax
from jax.experimental import pallas as pl
from jax.experimental.pallas import tpu as pltpu
from jax.experimental.pallas import tpu_sc as plsc

D_MODEL = 128
BATCH = 16384
SCALE = 0.2

_INFO = plsc.get_sparse_core_info()
_NC = _INFO.num_cores
_NS = _INFO.num_subcores
_LANES = _INFO.num_lanes
_NW = _NC * _NS
_B_PER_W = BATCH // _NW
_CHUNK = 64
_N_CHUNK = _B_PER_W // _CHUNK
_VPR = D_MODEL // _LANES


def _make_sc_gather():
    mesh = plsc.VectorSubcoreMesh(core_axis_name="c", subcore_axis_name="s")

    @functools.partial(
        pl.kernel,
        mesh=mesh,
        out_type=jax.ShapeDtypeStruct((BATCH, D_MODEL), jnp.float32),
        scratch_types=[
            pltpu.VMEM((_B_PER_W,), jnp.int32),
            pltpu.VMEM((_N_CHUNK, _CHUNK, D_MODEL), jnp.float32),
        ]
        + [pltpu.SemaphoreType.DMA] * (_N_CHUNK + 1),
    )
    def sc_gather(table_hbm, idx_hbm, out_hbm, idx_v, rows_v, *sems):
        gsems, ssem = sems[:_N_CHUNK], sems[_N_CHUNK]
        wid = lax.axis_index("s") * _NC + lax.axis_index("c")
        base = wid * _B_PER_W
        pltpu.sync_copy(idx_hbm.at[pl.ds(wid * _B_PER_W, _B_PER_W)], idx_v)
        gathers = [
            pltpu.async_copy(
                table_hbm.at[idx_v.at[pl.ds(j * _CHUNK, _CHUNK)]],
                rows_v.at[j],
                gsems[j],
            )
            for j in range(_N_CHUNK)
        ]
        stores = []
        for j in range(_N_CHUNK):
            gathers[j].wait()

            def scale_rows(r, _, j=j):
                for rr in range(2):
                    for c in range(_VPR):
                        sl = pl.ds(c * _LANES, _LANES)
                        rows_v[j, r * 2 + rr, sl] = rows_v[j, r * 2 + rr, sl] * SCALE
                return _

            lax.fori_loop(0, _CHUNK // 2, scale_rows, None)
            stores.append(
                pltpu.async_copy(
                    rows_v.at[j], out_hbm.at[pl.ds(base + j * _CHUNK, _CHUNK)], ssem
                )
            )
        for s in stores:
            s.wait()

    return sc_gather


_SC_GATHER = _make_sc_gather()


def kernel(pe, t):
    return _SC_GATHER(pe, t)
